# Initial kernel scaffold; baseline (speedup 1.0000x reference)
import jax
import jax.numpy as jnp
from jax.experimental import pallas as pl


def kernel(x, table):
    raise NotImplementedError("write your pallas kernel here")


# speedup vs baseline: 1.6696x; 1.6696x over previous
import functools

import jax
import jax.numpy as jnp
from jax import lax
from jax.experimental import pallas as pl
from jax.experimental.pallas import tpu as pltpu
from jax.experimental.pallas import tpu_sc as plsc

NUM_EMB = 100000
D = 64
B = 4096
F = 26

NC = 2
NS = 16
NW = NC * NS
BPW = B // NW
MC = 8
MB = BPW // MC
ROWS = MB * F
NSUB = 4
SUBW = ROWS // NSUB

_mesh = plsc.VectorSubcoreMesh(
    core_axis_name="c", subcore_axis_name="s", num_cores=NC, num_subcores=NS
)


@functools.partial(
    pl.kernel,
    out_type=jax.ShapeDtypeStruct((B, D), jnp.float32),
    mesh=_mesh,
    scratch_types=[
        pltpu.VMEM((MC, NSUB, SUBW), jnp.int32),
        pltpu.VMEM((ROWS, D), jnp.float32),
        pltpu.VMEM((ROWS, D), jnp.float32),
        pltpu.VMEM((BPW, D), jnp.float32),
        pltpu.SemaphoreType.DMA,
    ],
    compiler_params=pltpu.CompilerParams(use_tc_tiling_on_sc=False),
)
def _field_embed(x_hbm, table_hbm, out_hbm, idx_v, buf0, buf1, out_v, sem):
    wid = lax.axis_index("s") * NC + lax.axis_index("c")
    pltpu.sync_copy(x_hbm.at[wid], idx_v)

    bufs = (buf0, buf1)

    def start_gather(m, buf):
        return [
            pltpu.async_copy(
                table_hbm.at[idx_v.at[m, sub]],
                buf.at[pl.ds(sub * SUBW, SUBW)],
                sem,
            )
            for sub in range(NSUB)
        ]

    copies = start_gather(0, bufs[0])
    for m in range(MC):
        buf = bufs[m % 2]
        for cp in copies:
            cp.wait()
        if m + 1 < MC:
            copies = start_gather(m + 1, bufs[(m + 1) % 2])

        def pool_row(b, _, buf=buf, m=m):
            base = b * F
            acc = [buf[base, pl.ds(d * 16, 16)] for d in range(D // 16)]
            for f in range(1, F):
                for d in range(D // 16):
                    acc[d] = acc[d] + buf[base + f, pl.ds(d * 16, 16)]
            row = m * MB + b
            for d in range(D // 16):
                out_v[row, pl.ds(d * 16, 16)] = acc[d]
            return 0

        lax.fori_loop(0, MB, pool_row, 0)

    pltpu.sync_copy(out_v, out_hbm.at[pl.ds(wid * BPW, BPW)])


def kernel(x, table):
    xr = x.astype(jnp.int32).reshape(NW, MC, NSUB, SUBW)
    return _field_embed(xr, table)
